# baseline (speedup 1.0000x reference)
import functools

import jax
import jax.numpy as jnp
from jax import lax
from jax.experimental import pallas as pl
from jax.experimental.pallas import tpu as pltpu
from jax.experimental.pallas import tpu_sc as plsc

BATCH = 4
SEQ = 2048
D_MODEL = 768

NUM_CORES = 2
NUM_SUBCORES = 16
NUM_WORKERS = NUM_CORES * NUM_SUBCORES
B_PER_W = BATCH * SEQ // NUM_WORKERS
W_PER_BATCH = SEQ // B_PER_W
CHUNK = 64
N_CHUNKS = B_PER_W // CHUNK

_mesh = plsc.VectorSubcoreMesh(core_axis_name="c", subcore_axis_name="s")


@functools.partial(
    pl.kernel,
    mesh=_mesh,
    out_type=jax.ShapeDtypeStruct((BATCH, SEQ, D_MODEL), jnp.float32),
    scratch_types=[
        pltpu.VMEM((B_PER_W,), jnp.int32),
        pltpu.VMEM((2, CHUNK, D_MODEL), jnp.float32),
        pltpu.SemaphoreType.DMA,
        pltpu.SemaphoreType.DMA,
        pltpu.SemaphoreType.DMA,
        pltpu.SemaphoreType.DMA,
    ],
)
def _emb_lookup(
    idx_hbm, table_hbm, out_hbm, idx_v, rows_v, gsem, wsem0, wsem1, isem
):
    wid = lax.axis_index("s") * NUM_CORES + lax.axis_index("c")
    b = wid // W_PER_BATCH
    off = (wid % W_PER_BATCH) * B_PER_W
    half = B_PER_W // 2
    pltpu.sync_copy(idx_hbm.at[b, pl.ds(off, half)], idx_v.at[pl.ds(0, half)])
    rest = pltpu.async_copy(
        idx_hbm.at[b, pl.ds(off + half, half)],
        idx_v.at[pl.ds(half, half)],
        isem,
    )
    wsems = [wsem0, wsem1]
    writes = [None] * N_CHUNKS
    gather = pltpu.async_copy(
        table_hbm.at[idx_v.at[pl.ds(0, CHUNK)]], rows_v.at[0], gsem
    )
    for c in range(1, N_CHUNKS + 1):
        if c == N_CHUNKS // 2:
            rest.wait()
        p = c - 1
        gather.wait()
        writes[p] = pltpu.async_copy(
            rows_v.at[p % 2],
            out_hbm.at[b, pl.ds(off + p * CHUNK, CHUNK)],
            wsems[p % 2],
        )
        if c < N_CHUNKS:
            if c >= 2:
                writes[c - 2].wait()
            gather = pltpu.async_copy(
                table_hbm.at[idx_v.at[pl.ds(c * CHUNK, CHUNK)]],
                rows_v.at[c % 2],
                gsem,
            )
    writes[N_CHUNKS - 2].wait()
    writes[N_CHUNKS - 1].wait()


def kernel(input_ids, emb_table):
    return _emb_lookup(input_ids.astype(jnp.int32), emb_table)


# speedup vs baseline: 1.0500x; 1.0500x over previous
import functools

import jax
import jax.numpy as jnp
from jax import lax
from jax.experimental import pallas as pl
from jax.experimental.pallas import tpu as pltpu
from jax.experimental.pallas import tpu_sc as plsc

BATCH = 4
SEQ = 2048
D_MODEL = 768

NUM_CORES = 2
NUM_SUBCORES = 16
NUM_WORKERS = NUM_CORES * NUM_SUBCORES
B_PER_W = BATCH * SEQ // NUM_WORKERS
W_PER_BATCH = SEQ // B_PER_W
CHUNK = 64
N_CHUNKS = B_PER_W // CHUNK

_mesh = plsc.VectorSubcoreMesh(core_axis_name="c", subcore_axis_name="s")


@functools.partial(
    pl.kernel,
    mesh=_mesh,
    out_type=jax.ShapeDtypeStruct((BATCH, SEQ, D_MODEL), jnp.float32),
    scratch_types=[
        pltpu.VMEM((B_PER_W,), jnp.int32),
        pltpu.VMEM((2, CHUNK, D_MODEL), jnp.float32),
        pltpu.SemaphoreType.DMA,
        pltpu.SemaphoreType.DMA,
        pltpu.SemaphoreType.DMA,
    ],
)
def _emb_lookup(idx_hbm, table_hbm, out_hbm, idx_v, rows_v, gsem, wsem, isem):
    wid = lax.axis_index("s") * NUM_CORES + lax.axis_index("c")
    b = wid // W_PER_BATCH
    off = (wid % W_PER_BATCH) * B_PER_W
    half = B_PER_W // 2
    pltpu.sync_copy(idx_hbm.at[b, pl.ds(off, half)], idx_v.at[pl.ds(0, half)])
    rest = pltpu.async_copy(
        idx_hbm.at[b, pl.ds(off + half, half)],
        idx_v.at[pl.ds(half, half)],
        isem,
    )
    gathers = [None] * N_CHUNKS
    writes = [None] * N_CHUNKS
    for c in range(N_CHUNKS):
        if c == N_CHUNKS // 2:
            rest.wait()
        if c >= 2:
            writes[c - 2].wait()
        gathers[c] = pltpu.async_copy(
            table_hbm.at[idx_v.at[pl.ds(c * CHUNK, CHUNK)]],
            rows_v.at[c % 2],
            gsem,
        )
        if c >= 1:
            p = c - 1
            gathers[p].wait()
            writes[p] = pltpu.async_copy(
                rows_v.at[p % 2],
                out_hbm.at[b, pl.ds(off + p * CHUNK, CHUNK)],
                wsem,
            )
    last = N_CHUNKS - 1
    gathers[last].wait()
    writes[last] = pltpu.async_copy(
        rows_v.at[last % 2],
        out_hbm.at[b, pl.ds(off + last * CHUNK, CHUNK)],
        wsem,
    )
    writes[last - 1].wait()
    writes[last].wait()


def kernel(input_ids, emb_table):
    return _emb_lookup(input_ids.astype(jnp.int32), emb_table)
